# baseline (speedup 1.0000x reference)
import functools

import jax
import jax.numpy as jnp
from jax import lax
from jax.experimental import pallas as pl
from jax.experimental.pallas import tpu as pltpu
from jax.experimental.pallas import tpu_sc as plsc

B, N, D = 8, 576, 64
NTOK = B * N
K = 8192
TM = 512
GRID = NTOK // TM

NC, NS = 2, 16
NW = NC * NS
BPW = NTOK // NW
HALF = BPW // 2


def _argmin_body(z_ref, embT_ref, idx_ref, dsum_ref, enorm_ref):
    step = pl.program_id(0)

    @pl.when(step == 0)
    def _():
        e = embT_ref[...]
        enorm_ref[...] = jnp.sum(e * e, axis=0, keepdims=True)

    z = z_ref[...]
    dot = jnp.dot(z, embT_ref[...], preferred_element_type=jnp.float32)
    znorm = jnp.sum(z * z, axis=1, keepdims=True)
    d = (znorm + enorm_ref[...]) - 2.0 * dot
    m = jnp.min(d, axis=1, keepdims=True)
    iota = lax.broadcasted_iota(jnp.int32, (TM, K), 1)
    idx = jnp.min(jnp.where(d == m, iota, K), axis=1)
    idx_ref[...] = idx

    part = jnp.sum(m, keepdims=True).reshape(1, 1)
    prev = jnp.where(step == 0, jnp.zeros((1, 1), jnp.float32), dsum_ref[...])
    dsum_ref[...] = prev + part


_argmin_call = pl.pallas_call(
    _argmin_body,
    grid=(GRID,),
    in_specs=[
        pl.BlockSpec((TM, D), lambda i: (i, 0)),
        pl.BlockSpec((D, K), lambda i: (0, 0)),
    ],
    out_specs=[
        pl.BlockSpec((TM,), lambda i: (i,)),
        pl.BlockSpec((1, 1), lambda i: (0, 0)),
    ],
    out_shape=[
        jax.ShapeDtypeStruct((NTOK,), jnp.int32),
        jax.ShapeDtypeStruct((1, 1), jnp.float32),
    ],
    scratch_shapes=[pltpu.VMEM((1, K), jnp.float32)],
)


def _gather_body(table_hbm, idx_hbm, out_hbm, idx_v, rows_v, sem):
    wid = lax.axis_index("s") * NC + lax.axis_index("c")
    base = wid * BPW
    pltpu.sync_copy(idx_hbm.at[pl.ds(base, BPW)], idx_v)
    c0 = pltpu.async_copy(table_hbm.at[idx_v.at[pl.ds(0, HALF)]],
                          rows_v.at[pl.ds(0, HALF)], sem)
    c1 = pltpu.async_copy(table_hbm.at[idx_v.at[pl.ds(HALF, HALF)]],
                          rows_v.at[pl.ds(HALF, HALF)], sem)
    c0.wait()
    c1.wait()
    pltpu.sync_copy(rows_v, out_hbm.at[pl.ds(base, BPW)])


@functools.cache
def _gather_call():
    return pl.kernel(
        _gather_body,
        out_type=jax.ShapeDtypeStruct((NTOK, D), jnp.float32),
        mesh=plsc.VectorSubcoreMesh(core_axis_name="c", subcore_axis_name="s"),
        scratch_types=[
            pltpu.VMEM((BPW,), jnp.int32),
            pltpu.VMEM((BPW, D), jnp.float32),
            pltpu.SemaphoreType.DMA,
        ],
        compiler_params=pltpu.CompilerParams(use_tc_tiling_on_sc=False),
    )


def kernel(z, emb_weight):
    z = z.astype(jnp.float32)
    zf = z.reshape(NTOK, D)
    idx_flat, dsum = _argmin_call(zf, emb_weight.T)
    quantized = _gather_call()(emb_weight, idx_flat).reshape(z.shape)
    mse = dsum.reshape(()) / float(NTOK * D)
    loss = 1.25 * mse
    zero = jnp.zeros((), jnp.float32)
    q_indices = idx_flat.reshape(B, N)
    return (z, emb_weight, quantized, q_indices, loss, mse, mse,
            zero, zero, zero)


# speedup vs baseline: 1.0187x; 1.0187x over previous
import functools

import jax
import jax.numpy as jnp
from jax import lax
from jax.experimental import pallas as pl
from jax.experimental.pallas import tpu as pltpu
from jax.experimental.pallas import tpu_sc as plsc

B, N, D = 8, 576, 64
NTOK = B * N
K = 8192
TM = 512
GRID = NTOK // TM

NC, NS = 2, 16
NW = NC * NS
BPW = NTOK // NW
HALF = BPW // 2


def _argmin_body(z_ref, embT2_ref, idx_ref, dsum_ref, enorm_ref):
    step = pl.program_id(0)

    @pl.when(step == 0)
    def _():
        e2 = embT2_ref[...]
        enorm_ref[...] = 0.25 * jnp.sum(e2 * e2, axis=0, keepdims=True)

    z = z_ref[...]
    dot2 = jnp.dot(z, embT2_ref[...], preferred_element_type=jnp.float32)
    znorm = jnp.sum(z * z, axis=1, keepdims=True)
    d = (znorm + enorm_ref[...]) - dot2
    m = jnp.min(d, axis=1, keepdims=True)
    idx_ref[...] = jnp.argmin(d, axis=1).astype(jnp.int32)

    part = jnp.sum(m, keepdims=True).reshape(1, 1)
    prev = jnp.where(step == 0, jnp.zeros((1, 1), jnp.float32), dsum_ref[...])
    dsum_ref[...] = prev + part


_argmin_call = pl.pallas_call(
    _argmin_body,
    grid=(GRID,),
    in_specs=[
        pl.BlockSpec((TM, D), lambda i: (i, 0)),
        pl.BlockSpec((D, K), lambda i: (0, 0)),
    ],
    out_specs=[
        pl.BlockSpec((TM,), lambda i: (i,)),
        pl.BlockSpec((1, 1), lambda i: (0, 0)),
    ],
    out_shape=[
        jax.ShapeDtypeStruct((NTOK,), jnp.int32),
        jax.ShapeDtypeStruct((1, 1), jnp.float32),
    ],
    scratch_shapes=[pltpu.VMEM((1, K), jnp.float32)],
)


def _gather_body(table_hbm, idx_hbm, out_hbm, idx_v, rows_v, sem):
    wid = lax.axis_index("s") * NC + lax.axis_index("c")
    base = wid * BPW
    pltpu.sync_copy(idx_hbm.at[pl.ds(base, BPW)], idx_v)
    c0 = pltpu.async_copy(table_hbm.at[idx_v.at[pl.ds(0, HALF)]],
                          rows_v.at[pl.ds(0, HALF)], sem)
    c1 = pltpu.async_copy(table_hbm.at[idx_v.at[pl.ds(HALF, HALF)]],
                          rows_v.at[pl.ds(HALF, HALF)], sem)
    c0.wait()
    c1.wait()
    pltpu.sync_copy(rows_v, out_hbm.at[pl.ds(base, BPW)])


@functools.cache
def _gather_call():
    return pl.kernel(
        _gather_body,
        out_type=jax.ShapeDtypeStruct((NTOK, D), jnp.float32),
        mesh=plsc.VectorSubcoreMesh(core_axis_name="c", subcore_axis_name="s"),
        scratch_types=[
            pltpu.VMEM((BPW,), jnp.int32),
            pltpu.VMEM((BPW, D), jnp.float32),
            pltpu.SemaphoreType.DMA,
        ],
        compiler_params=pltpu.CompilerParams(use_tc_tiling_on_sc=False),
    )


def kernel(z, emb_weight):
    z = z.astype(jnp.float32)
    zf = z.reshape(NTOK, D)
    idx_flat, dsum = _argmin_call(zf, (emb_weight * 2.0).T)
    quantized = _gather_call()(emb_weight, idx_flat).reshape(z.shape)
    mse = dsum.reshape(()) / float(NTOK * D)
    loss = 1.25 * mse
    zero = jnp.zeros((), jnp.float32)
    q_indices = idx_flat.reshape(B, N)
    return (z, emb_weight, quantized, q_indices, loss, mse, mse,
            zero, zero, zero)
